# Initial kernel scaffold; baseline (speedup 1.0000x reference)
import jax
import jax.numpy as jnp
from jax.experimental import pallas as pl


def kernel(x, prompt, prompt_key):
    raise NotImplementedError("write your pallas kernel here")


# speedup vs baseline: 1.0295x; 1.0295x over previous
import jax
import jax.numpy as jnp
from jax import lax
from jax.experimental import pallas as pl

B = 128
P = 10
K = 5
LP = 5
D = 1000
BLK = 8
GRID = B // BLK
PE_W = (K * LP + 1) * D


def _l2n(v):
    return v * lax.rsqrt(jnp.maximum(jnp.sum(v * v, axis=1, keepdims=True), 1e-12))


def _body(x_ref, pf_ref, pk_ref,
          idx_ref, pn_ref, xn_ref, sim_ref, bkn_ref, rs_ref, pe_ref):
    i = pl.program_id(0)
    x = x_ref[...]
    pk = pk_ref[...]
    pf = pf_ref[...]

    pn = _l2n(pk)
    xn = _l2n(x)
    sim = lax.dot_general(xn, pn, (((1,), (1,)), ((), ())))

    colj = lax.broadcasted_iota(jnp.int32, (1, P), 1)
    rank = jnp.zeros((B, P), jnp.int32)
    for jp in range(P):
        sj = sim[:, jp:jp + 1]
        gt = (sj > sim).astype(jnp.int32)
        eq = (sj == sim).astype(jnp.int32) * (colj > jp).astype(jnp.int32)
        rank = rank + gt + eq
    in_top = (rank < K).astype(jnp.int32)
    counts = jnp.sum(in_top, axis=0, keepdims=True)

    crank = jnp.zeros((1, P), jnp.int32)
    for jp in range(P):
        cj = counts[:, jp:jp + 1]
        gt = (cj > counts).astype(jnp.int32)
        eq = (cj == counts).astype(jnp.int32) * (colj > jp).astype(jnp.int32)
        crank = crank + gt + eq
    sel = crank < K
    self32 = sel.astype(jnp.float32)

    r_io = lax.broadcasted_iota(jnp.int32, (P, P), 0)
    c_io = lax.broadcasted_iota(jnp.int32, (P, P), 1)
    strict_lt = (r_io < c_io).astype(jnp.float32)
    pos = lax.dot_general(self32, strict_lt, (((1,), (0,)), ((), ())))

    s_io = lax.broadcasted_iota(jnp.int32, (K, P), 0).astype(jnp.float32)
    oh = ((s_io == pos) & sel).astype(jnp.float32)

    coljf = colj.astype(jnp.float32)
    major_f = lax.dot_general(coljf, oh, (((1,), (1,)), ((), ())))
    major_i = major_f.astype(jnp.int32)

    sel_key = lax.dot_general(oh, pn, (((1,), (0,)), ((), ())))
    sel_pr = lax.dot_general(oh, pf, (((1,), (0,)), ((), ())))

    key_row = jnp.concatenate([sel_key[s:s + 1, :] for s in range(K)], axis=1)
    pr_row = jnp.concatenate([sel_pr[s:s + 1, :] for s in range(K)], axis=1)

    x_blk = x_ref[pl.ds(i * BLK, BLK), :]
    xn_blk = _l2n(x_blk)
    sim_blk = lax.dot_general(xn_blk, pn, (((1,), (1,)), ((), ())))

    idx_ref[...] = jnp.broadcast_to(major_i, (BLK, K))
    xn_ref[...] = xn_blk
    sim_ref[...] = sim_blk
    bkn_ref[...] = jnp.broadcast_to(key_row, (BLK, K * D))
    pe_ref[...] = jnp.concatenate(
        [jnp.broadcast_to(pr_row, (BLK, K * LP * D)), x_blk], axis=1)

    @pl.when(i == 0)
    def _():
        pn_ref[...] = pn
        ksum = jnp.sum(sel_key, axis=0, keepdims=True)
        xnsum = jnp.sum(xn, axis=0, keepdims=True)
        rs_ref[...] = (jnp.sum(ksum * xnsum) / B).reshape(1, 1)


def kernel(x, prompt, prompt_key):
    pf = prompt.reshape(P, LP * D)
    outs = pl.pallas_call(
        _body,
        grid=(GRID,),
        in_specs=[
            pl.BlockSpec((B, D), lambda i: (0, 0)),
            pl.BlockSpec((P, LP * D), lambda i: (0, 0)),
            pl.BlockSpec((P, D), lambda i: (0, 0)),
        ],
        out_specs=[
            pl.BlockSpec((BLK, K), lambda i: (i, 0)),
            pl.BlockSpec((P, D), lambda i: (0, 0)),
            pl.BlockSpec((BLK, D), lambda i: (i, 0)),
            pl.BlockSpec((BLK, P), lambda i: (i, 0)),
            pl.BlockSpec((BLK, K * D), lambda i: (i, 0)),
            pl.BlockSpec((1, 1), lambda i: (0, 0)),
            pl.BlockSpec((BLK, PE_W), lambda i: (i, 0)),
        ],
        out_shape=[
            jax.ShapeDtypeStruct((B, K), jnp.int32),
            jax.ShapeDtypeStruct((P, D), jnp.float32),
            jax.ShapeDtypeStruct((B, D), jnp.float32),
            jax.ShapeDtypeStruct((B, P), jnp.float32),
            jax.ShapeDtypeStruct((B, K * D), jnp.float32),
            jax.ShapeDtypeStruct((1, 1), jnp.float32),
            jax.ShapeDtypeStruct((B, PE_W), jnp.float32),
        ],
    )(x, pf, prompt_key)
    idx_b, pn, xn, sim, bkn, rs, pe = outs
    return (idx_b, pn, xn, sim, bkn.reshape(B, K, D), rs[0, 0], pe)


# speedup vs baseline: 1.4337x; 1.3927x over previous
import jax
import jax.numpy as jnp
from jax import lax
from jax.experimental import pallas as pl
from jax.experimental.pallas import tpu as pltpu

B = 128
P = 10
K = 5
LP = 5
D = 1000
BLK = 16
GRID = B // BLK
PE_W = (K * LP + 1) * D


def _l2n(v):
    return v * lax.rsqrt(jnp.maximum(jnp.sum(v * v, axis=1, keepdims=True), 1e-12))


def _body(x_ref, pf_ref, pk_ref,
          idx_ref, pn_ref, xn_ref, sim_ref, bkn_ref, rs_ref, pe_ref,
          prow_ref, krow_ref, major_ref, pns_ref):
    i = pl.program_id(0)

    @pl.when(i == 0)
    def _():
        x = x_ref[...]
        pk = pk_ref[...]
        pf = pf_ref[...]

        pn = _l2n(pk)
        xn = _l2n(x)
        sim = lax.dot_general(xn, pn, (((1,), (1,)), ((), ())))

        colj = lax.broadcasted_iota(jnp.int32, (1, P), 1)
        rank = jnp.zeros((B, P), jnp.int32)
        for jp in range(P):
            sj = sim[:, jp:jp + 1]
            gt = (sj > sim).astype(jnp.int32)
            eq = (sj == sim).astype(jnp.int32) * (colj > jp).astype(jnp.int32)
            rank = rank + gt + eq
        in_top = (rank < K).astype(jnp.int32)
        counts = jnp.sum(in_top, axis=0, keepdims=True)

        crank = jnp.zeros((1, P), jnp.int32)
        for jp in range(P):
            cj = counts[:, jp:jp + 1]
            gt = (cj > counts).astype(jnp.int32)
            eq = (cj == counts).astype(jnp.int32) * (colj > jp).astype(jnp.int32)
            crank = crank + gt + eq
        sel = crank < K
        self32 = sel.astype(jnp.float32)

        r_io = lax.broadcasted_iota(jnp.int32, (P, P), 0)
        c_io = lax.broadcasted_iota(jnp.int32, (P, P), 1)
        strict_lt = (r_io < c_io).astype(jnp.float32)
        pos = lax.dot_general(self32, strict_lt, (((1,), (0,)), ((), ())))

        s_io = lax.broadcasted_iota(jnp.int32, (K, P), 0).astype(jnp.float32)
        oh = ((s_io == pos) & sel).astype(jnp.float32)

        coljf = colj.astype(jnp.float32)
        major_f = lax.dot_general(coljf, oh, (((1,), (1,)), ((), ())))

        sel_key = lax.dot_general(oh, pn, (((1,), (0,)), ((), ())))
        sel_pr = lax.dot_general(oh, pf, (((1,), (0,)), ((), ())))

        krow_ref[...] = jnp.concatenate(
            [sel_key[s:s + 1, :] for s in range(K)], axis=1)
        prow_ref[...] = jnp.concatenate(
            [sel_pr[s:s + 1, :] for s in range(K)], axis=1)
        major_ref[...] = major_f.astype(jnp.int32)
        pns_ref[...] = pn
        pn_ref[...] = pn

        ksum = jnp.sum(sel_key, axis=0, keepdims=True)
        xnsum = jnp.sum(xn, axis=0, keepdims=True)
        rs_ref[...] = (jnp.sum(ksum * xnsum) / B).reshape(1, 1)

    x_blk = x_ref[pl.ds(i * BLK, BLK), :]
    xn_blk = _l2n(x_blk)
    pn = pns_ref[...]
    sim_blk = lax.dot_general(xn_blk, pn, (((1,), (1,)), ((), ())))

    idx_ref[...] = jnp.broadcast_to(major_ref[...], (BLK, K))
    xn_ref[...] = xn_blk
    sim_ref[...] = sim_blk
    bkn_ref[...] = jnp.broadcast_to(krow_ref[...], (BLK, K * D))
    pe_ref[...] = jnp.concatenate(
        [jnp.broadcast_to(prow_ref[...], (BLK, K * LP * D)), x_blk], axis=1)


def kernel(x, prompt, prompt_key):
    pf = prompt.reshape(P, LP * D)
    outs = pl.pallas_call(
        _body,
        grid=(GRID,),
        in_specs=[
            pl.BlockSpec((B, D), lambda i: (0, 0)),
            pl.BlockSpec((P, LP * D), lambda i: (0, 0)),
            pl.BlockSpec((P, D), lambda i: (0, 0)),
        ],
        out_specs=[
            pl.BlockSpec((BLK, K), lambda i: (i, 0)),
            pl.BlockSpec((P, D), lambda i: (0, 0)),
            pl.BlockSpec((BLK, D), lambda i: (i, 0)),
            pl.BlockSpec((BLK, P), lambda i: (i, 0)),
            pl.BlockSpec((BLK, K * D), lambda i: (i, 0)),
            pl.BlockSpec((1, 1), lambda i: (0, 0)),
            pl.BlockSpec((BLK, PE_W), lambda i: (i, 0)),
        ],
        out_shape=[
            jax.ShapeDtypeStruct((B, K), jnp.int32),
            jax.ShapeDtypeStruct((P, D), jnp.float32),
            jax.ShapeDtypeStruct((B, D), jnp.float32),
            jax.ShapeDtypeStruct((B, P), jnp.float32),
            jax.ShapeDtypeStruct((B, K * D), jnp.float32),
            jax.ShapeDtypeStruct((1, 1), jnp.float32),
            jax.ShapeDtypeStruct((B, PE_W), jnp.float32),
        ],
        scratch_shapes=[
            pltpu.VMEM((1, K * LP * D), jnp.float32),
            pltpu.VMEM((1, K * D), jnp.float32),
            pltpu.VMEM((1, K), jnp.int32),
            pltpu.VMEM((P, D), jnp.float32),
        ],
    )(x, pf, prompt_key)
    idx_b, pn, xn, sim, bkn, rs, pe = outs
    return (idx_b, pn, xn, sim, bkn.reshape(B, K, D), rs[0, 0], pe)


# speedup vs baseline: 1.4429x; 1.0064x over previous
import jax
import jax.numpy as jnp
from jax import lax
from jax.experimental import pallas as pl
from jax.experimental.pallas import tpu as pltpu

B = 128
P = 10
K = 5
LP = 5
D = 1000
BLK = 32
GRID = B // BLK
PE_W = (K * LP + 1) * D


def _l2n(v):
    return v * lax.rsqrt(jnp.maximum(jnp.sum(v * v, axis=1, keepdims=True), 1e-12))


def _body(x_ref, pf_ref, pk_ref,
          idx_ref, pn_ref, xn_ref, sim_ref, bkn_ref, rs_ref, pe_ref,
          prow_ref, krow_ref, major_ref, pns_ref):
    i = pl.program_id(0)

    @pl.when(i == 0)
    def _():
        x = x_ref[...]
        pk = pk_ref[...]
        pf = pf_ref[...]

        pn = _l2n(pk)
        xn = _l2n(x)
        sim = lax.dot_general(xn, pn, (((1,), (1,)), ((), ())))

        colj = lax.broadcasted_iota(jnp.int32, (1, P), 1)
        rank = jnp.zeros((B, P), jnp.int32)
        for jp in range(P):
            sj = sim[:, jp:jp + 1]
            gt = (sj > sim).astype(jnp.int32)
            eq = (sj == sim).astype(jnp.int32) * (colj > jp).astype(jnp.int32)
            rank = rank + gt + eq
        in_top = (rank < K).astype(jnp.int32)
        counts = jnp.sum(in_top, axis=0, keepdims=True)

        crank = jnp.zeros((1, P), jnp.int32)
        for jp in range(P):
            cj = counts[:, jp:jp + 1]
            gt = (cj > counts).astype(jnp.int32)
            eq = (cj == counts).astype(jnp.int32) * (colj > jp).astype(jnp.int32)
            crank = crank + gt + eq
        sel = crank < K
        self32 = sel.astype(jnp.float32)

        r_io = lax.broadcasted_iota(jnp.int32, (P, P), 0)
        c_io = lax.broadcasted_iota(jnp.int32, (P, P), 1)
        strict_lt = (r_io < c_io).astype(jnp.float32)
        pos = lax.dot_general(self32, strict_lt, (((1,), (0,)), ((), ())))

        s_io = lax.broadcasted_iota(jnp.int32, (K, P), 0).astype(jnp.float32)
        oh = ((s_io == pos) & sel).astype(jnp.float32)

        coljf = colj.astype(jnp.float32)
        major_f = lax.dot_general(coljf, oh, (((1,), (1,)), ((), ())))

        sel_key = lax.dot_general(oh, pn, (((1,), (0,)), ((), ())))
        sel_pr = lax.dot_general(oh, pf, (((1,), (0,)), ((), ())))

        krow_ref[...] = jnp.concatenate(
            [sel_key[s:s + 1, :] for s in range(K)], axis=1)
        prow_ref[...] = jnp.concatenate(
            [sel_pr[s:s + 1, :] for s in range(K)], axis=1)
        major_ref[...] = major_f.astype(jnp.int32)
        pns_ref[...] = pn
        pn_ref[...] = pn

        ksum = jnp.sum(sel_key, axis=0, keepdims=True)
        xnsum = jnp.sum(xn, axis=0, keepdims=True)
        rs_ref[...] = (jnp.sum(ksum * xnsum) / B).reshape(1, 1)

    x_blk = x_ref[pl.ds(i * BLK, BLK), :]
    xn_blk = _l2n(x_blk)
    pn = pns_ref[...]
    sim_blk = lax.dot_general(xn_blk, pn, (((1,), (1,)), ((), ())))

    idx_ref[...] = jnp.broadcast_to(major_ref[...], (BLK, K))
    xn_ref[...] = xn_blk
    sim_ref[...] = sim_blk
    bkn_ref[...] = jnp.broadcast_to(krow_ref[...], (BLK, K * D))
    pe_ref[...] = jnp.concatenate(
        [jnp.broadcast_to(prow_ref[...], (BLK, K * LP * D)), x_blk], axis=1)


def kernel(x, prompt, prompt_key):
    pf = prompt.reshape(P, LP * D)
    outs = pl.pallas_call(
        _body,
        grid=(GRID,),
        in_specs=[
            pl.BlockSpec((B, D), lambda i: (0, 0)),
            pl.BlockSpec((P, LP * D), lambda i: (0, 0)),
            pl.BlockSpec((P, D), lambda i: (0, 0)),
        ],
        out_specs=[
            pl.BlockSpec((BLK, K), lambda i: (i, 0)),
            pl.BlockSpec((P, D), lambda i: (0, 0)),
            pl.BlockSpec((BLK, D), lambda i: (i, 0)),
            pl.BlockSpec((BLK, P), lambda i: (i, 0)),
            pl.BlockSpec((BLK, K * D), lambda i: (i, 0)),
            pl.BlockSpec((1, 1), lambda i: (0, 0)),
            pl.BlockSpec((BLK, PE_W), lambda i: (i, 0)),
        ],
        out_shape=[
            jax.ShapeDtypeStruct((B, K), jnp.int32),
            jax.ShapeDtypeStruct((P, D), jnp.float32),
            jax.ShapeDtypeStruct((B, D), jnp.float32),
            jax.ShapeDtypeStruct((B, P), jnp.float32),
            jax.ShapeDtypeStruct((B, K * D), jnp.float32),
            jax.ShapeDtypeStruct((1, 1), jnp.float32),
            jax.ShapeDtypeStruct((B, PE_W), jnp.float32),
        ],
        scratch_shapes=[
            pltpu.VMEM((1, K * LP * D), jnp.float32),
            pltpu.VMEM((1, K * D), jnp.float32),
            pltpu.VMEM((1, K), jnp.int32),
            pltpu.VMEM((P, D), jnp.float32),
        ],
    )(x, pf, prompt_key)
    idx_b, pn, xn, sim, bkn, rs, pe = outs
    return (idx_b, pn, xn, sim, bkn.reshape(B, K, D), rs[0, 0], pe)
